# baseline (speedup 1.0000x reference)
import functools

import jax
import jax.numpy as jnp
from jax import lax
from jax.experimental import pallas as pl
from jax.experimental.pallas import tpu as pltpu
from jax.experimental.pallas import tpu_sc as plsc

def _round8(v):
  return (v + 7) // 8 * 8


NC = 2
NS = 16
L = 16
CH = 128


def _sc_mesh():
  return plsc.VectorSubcoreMesh(core_axis_name="c", subcore_axis_name="s",
                                num_cores=NC, num_subcores=NS)


def _deg_call(N, E):
  nw = NC * NS
  epw = E // nw
  rpt = _round8(-(-N // NS))
  NP = rpt * NS

  @functools.partial(
      pl.kernel,
      out_type=jax.ShapeDtypeStruct((nw, NP), jnp.float32),
      mesh=_sc_mesh(),
      scratch_types=[
          pltpu.VMEM((epw,), jnp.int32),
          pltpu.VMEM((NP,), jnp.float32),
      ],
      compiler_params=pltpu.CompilerParams(needs_layout_passes=False),
  )
  def k(col_hbm, degp_hbm, colstage, hist):
    cid = lax.axis_index("c")
    sid = lax.axis_index("s")
    wid = cid * NS + sid

    def zbody(i, c):
      hist[pl.ds(i * L, L)] = jnp.zeros(L, jnp.float32)
      return c

    lax.fori_loop(0, NP // L, zbody, 0)
    pltpu.sync_copy(col_hbm.at[pl.ds(wid * epw, epw)], colstage)
    ones = jnp.ones(L, jnp.float32)

    def body(i, c):
      idx = colstage[pl.ds(i * L, L)]
      plsc.addupdate_scatter(hist, [idx], ones)
      return c

    lax.fori_loop(0, epw // L, body, 0)
    pltpu.sync_copy(hist, degp_hbm.at[pl.ds(wid, 1)].at[0])

  return k


def _edge_call(N, nch, F):
  rpt = _round8(-(-N // NS))
  NP = rpt * NS
  R = 2
  assert nch % R == 0 and nch >= 2 * R

  @functools.partial(
      pl.kernel,
      out_type=jax.ShapeDtypeStruct((NC, NP, F), jnp.float32),
      mesh=_sc_mesh(),
      scratch_types=[
          [pltpu.VMEM((CH,), jnp.int32)] * R,
          [pltpu.VMEM((CH,), jnp.int32)] * R,
          pltpu.VMEM((R, CH, F), jnp.float32),
          pltpu.VMEM_SHARED((NP, F), jnp.float32),
          [pltpu.SemaphoreType.DMA] * R,
      ],
  )
  def k(xs_hbm, row_hbm, col_hbm, zeros_hbm, accp_hbm,
        rowb, colb, rows_v, acc, sem_g):
    cid = lax.axis_index("c")
    sid = lax.axis_index("s")
    wid = cid * NS + sid
    rbase = sid * rpt
    pltpu.sync_copy(zeros_hbm, acc.at[pl.ds(rbase, rpt)])
    plsc.subcore_barrier()
    ebase = wid * (nch * CH)

    def ix_load(i, ph):
      off = ebase + i * CH
      pltpu.sync_copy(row_hbm.at[pl.ds(off, CH)], rowb[ph])
      pltpu.sync_copy(col_hbm.at[pl.ds(off, CH)], colb[ph])

    def g_start(ph):
      pltpu.async_copy(xs_hbm.at[rowb[ph]], rows_v.at[ph], sem_g[ph])

    def g_wait(ph):
      pltpu.make_async_copy(xs_hbm.at[rowb[ph]], rows_v.at[ph],
                            sem_g[ph]).wait()

    def scat(ph):
      pltpu.sync_copy(rows_v.at[ph], acc.at[colb[ph]], add=True)

    def step(i, ph, prefetch=True):
      if prefetch:
        ix_load(i + 1, (ph + 1) % R)
        g_start((ph + 1) % R)
      g_wait(ph)
      scat(ph)

    ix_load(0, 0)
    g_start(0)

    def body(h, c):
      step(h * R + 0, 0)
      step(h * R + 1, 1)
      return c

    lax.fori_loop(0, nch // R - 1, body, 0)
    step(nch - 2, 0)
    step(nch - 1, 1, prefetch=False)
    plsc.subcore_barrier()
    pltpu.sync_copy(acc.at[pl.ds(rbase, rpt)],
                    accp_hbm.at[cid, pl.ds(rbase, rpt)])

  return k


def _b1_call(N, F, H, BN):
  def body(x_ref, w_ref, degp_ref, xs_ref, dinv_ref):
    deg = 1.0 + jnp.sum(degp_ref[...], axis=0)
    di = lax.rsqrt(deg)
    xw = jnp.dot(x_ref[...], w_ref[...], preferred_element_type=jnp.float32)
    xs_ref[...] = xw * di[:, None]
    dinv_ref[...] = di[:, None]

  return pl.pallas_call(
      body,
      grid=(pl.cdiv(N, BN),),
      in_specs=[
          pl.BlockSpec((BN, F), lambda i: (i, 0)),
          pl.BlockSpec((F, H), lambda i: (0, 0)),
          pl.BlockSpec((NC * NS, BN), lambda i: (0, i)),
      ],
      out_specs=[
          pl.BlockSpec((BN, H), lambda i: (i, 0)),
          pl.BlockSpec((BN, 1), lambda i: (i, 0)),
      ],
      out_shape=[
          jax.ShapeDtypeStruct((N, H), jnp.float32),
          jax.ShapeDtypeStruct((N, 1), jnp.float32),
      ],
  )


def _b2_call(N, H, O, BN):
  def body(accp_ref, xs1_ref, dinv_ref, b1_ref, w2_ref, xs2_ref):
    s = accp_ref[0] + accp_ref[1] + xs1_ref[...]
    h = jnp.maximum(s * dinv_ref[...] + b1_ref[...], 0.0)
    xs2_ref[...] = jnp.dot(h, w2_ref[...],
                           preferred_element_type=jnp.float32) * dinv_ref[...]

  return pl.pallas_call(
      body,
      grid=(pl.cdiv(N, BN),),
      in_specs=[
          pl.BlockSpec((NC, BN, H), lambda i: (0, i, 0)),
          pl.BlockSpec((BN, H), lambda i: (i, 0)),
          pl.BlockSpec((BN, 1), lambda i: (i, 0)),
          pl.BlockSpec((1, H), lambda i: (0, 0)),
          pl.BlockSpec((H, O), lambda i: (0, 0)),
      ],
      out_specs=pl.BlockSpec((BN, O), lambda i: (i, 0)),
      out_shape=jax.ShapeDtypeStruct((N, O), jnp.float32),
  )


def _b3_call(N, O, BN):
  def body(accp_ref, xs2_ref, dinv_ref, b2_ref, out_ref):
    s = accp_ref[0] + accp_ref[1] + xs2_ref[...]
    out_ref[...] = s * dinv_ref[...] + b2_ref[...]

  return pl.pallas_call(
      body,
      grid=(pl.cdiv(N, BN),),
      in_specs=[
          pl.BlockSpec((NC, BN, O), lambda i: (0, i, 0)),
          pl.BlockSpec((BN, O), lambda i: (i, 0)),
          pl.BlockSpec((BN, 1), lambda i: (i, 0)),
          pl.BlockSpec((1, O), lambda i: (0, 0)),
      ],
      out_specs=pl.BlockSpec((BN, O), lambda i: (i, 0)),
      out_shape=jax.ShapeDtypeStruct((N, O), jnp.float32),
  )


def kernel(x, edge_index, edge_attr, W1, b1, W2, b2):
  N, F = x.shape
  H = W1.shape[1]
  O = W2.shape[1]
  E = edge_index.shape[1]
  del edge_attr
  assert E % (NC * NS) == 0 and N % NS == 0
  BN = 1024

  row = edge_index[0]
  col = edge_index[1]
  rpt = _round8(-(-N // NS))
  zerosF = jnp.zeros((rpt, F), jnp.float32)

  nw = NC * NS
  nch = -(-E // (nw * CH))
  nch = (nch + 3) // 4 * 4
  E3 = nw * nch * CH
  row_f = jnp.concatenate([row, jnp.zeros((E3 - E,), jnp.int32)])
  col_f = jnp.concatenate([col, jnp.full((E3 - E,), N, jnp.int32)])

  degp = _deg_call(N, E)(col)
  xs1, dinv = _b1_call(N, F, H, BN)(x, W1, degp)
  accp1 = _edge_call(N, nch, H)(xs1, row_f, col_f, zerosF)
  xs2 = _b2_call(N, H, O, BN)(accp1, xs1, dinv, b1.reshape(1, H), W2)
  accp2 = _edge_call(N, nch, O)(xs2, row_f, col_f, zerosF)
  out = _b3_call(N, O, BN)(accp2, xs2, dinv, b2.reshape(1, O))
  return out


# speedup vs baseline: 3.0360x; 3.0360x over previous
import functools

import jax
import jax.numpy as jnp
from jax import lax
from jax.experimental import pallas as pl
from jax.experimental.pallas import tpu as pltpu
from jax.experimental.pallas import tpu_sc as plsc

def _round8(v):
  return (v + 7) // 8 * 8


NC = 2
NS = 16
L = 16
CH = 128


def _sc_mesh():
  return plsc.VectorSubcoreMesh(core_axis_name="c", subcore_axis_name="s",
                                num_cores=NC, num_subcores=NS)


def _deg_call(N, E):
  nw = NC * NS
  epw = E // nw
  rpt = _round8(-(-N // NS))
  NP = rpt * NS

  @functools.partial(
      pl.kernel,
      out_type=jax.ShapeDtypeStruct((nw, NP), jnp.float32),
      mesh=_sc_mesh(),
      scratch_types=[
          pltpu.VMEM((epw,), jnp.int32),
          pltpu.VMEM((NP,), jnp.float32),
      ],
      compiler_params=pltpu.CompilerParams(needs_layout_passes=False),
  )
  def k(col_hbm, degp_hbm, colstage, hist):
    cid = lax.axis_index("c")
    sid = lax.axis_index("s")
    wid = cid * NS + sid

    def zbody(i, c):
      hist[pl.ds(i * L, L)] = jnp.zeros(L, jnp.float32)
      return c

    lax.fori_loop(0, NP // L, zbody, 0)
    pltpu.sync_copy(col_hbm.at[pl.ds(wid * epw, epw)], colstage)
    ones = jnp.ones(L, jnp.float32)

    def body(i, c):
      idx = colstage[pl.ds(i * L, L)]
      plsc.addupdate_scatter(hist, [idx], ones)
      return c

    lax.fori_loop(0, epw // L, body, 0)
    pltpu.sync_copy(hist, degp_hbm.at[pl.ds(wid, 1)].at[0])

  return k


def _edge_call(N, nch, F):
  rpt = _round8(-(-N // NS))
  NP = rpt * NS
  R = 2
  RX = 4
  assert nch % 4 == 0 and nch >= 8

  @functools.partial(
      pl.kernel,
      out_type=jax.ShapeDtypeStruct((NC, NP, F), jnp.float32),
      mesh=_sc_mesh(),
      scratch_types=[
          pltpu.VMEM((RX, 2, CH), jnp.int32),
          [pltpu.VMEM((CH, F), jnp.float32)] * R,
          pltpu.VMEM_SHARED((NP, F), jnp.float32),
          [pltpu.SemaphoreType.DMA] * RX,
          [pltpu.SemaphoreType.DMA] * R,
      ],
  )
  def k(xs_hbm, idx3_hbm, zeros_hbm, accp_hbm,
        idx_v, rows_v, acc, sem_ix, sem_g):
    cid = lax.axis_index("c")
    sid = lax.axis_index("s")
    wid = cid * NS + sid
    rbase = sid * rpt
    pltpu.sync_copy(zeros_hbm, acc.at[pl.ds(rbase, rpt)])
    plsc.subcore_barrier()

    def ix_start(i, s):
      pltpu.async_copy(idx3_hbm.at[wid, i], idx_v.at[s], sem_ix[s])

    def ix_wait(i, s):
      pltpu.make_async_copy(idx3_hbm.at[wid, i], idx_v.at[s],
                            sem_ix[s]).wait()

    def g_start(s, b):
      return pltpu.async_copy(xs_hbm.at[idx_v.at[s, 0]], rows_v[b], sem_g[b])

    def scat(s, b):
      pltpu.sync_copy(rows_v[b], acc.at[idx_v.at[s, 1]], add=True)

    def step(i, ph, ixfetch=True):
      if ixfetch:
        ix_start(i + 2, (ph + 2) % RX)
      ix_wait(i + 1, (ph + 1) % RX)
      d = g_start((ph + 1) % RX, (ph + 1) % R)
      scat(ph % RX, ph % R)
      d.wait()

    ix_start(0, 0)
    ix_start(1, 1)
    ix_wait(0, 0)
    g_start(0, 0).wait()

    def body(h, c):
      for ph in range(4):
        step(h * 4 + ph, ph)
      return c

    lax.fori_loop(0, nch // 4 - 1, body, 0)
    i0 = nch - 4
    step(i0 + 0, 0)
    step(i0 + 1, 1)
    step(i0 + 2, 2, ixfetch=False)
    scat(3, 1)
    plsc.subcore_barrier()
    pltpu.sync_copy(acc.at[pl.ds(rbase, rpt)],
                    accp_hbm.at[cid, pl.ds(rbase, rpt)])

  return k


def _b1_call(N, F, H, BN):
  def body(x_ref, w_ref, degp_ref, xs_ref, dinv_ref):
    deg = 1.0 + jnp.sum(degp_ref[...], axis=0)
    di = lax.rsqrt(deg)
    xw = jnp.dot(x_ref[...], w_ref[...], preferred_element_type=jnp.float32)
    xs_ref[...] = xw * di[:, None]
    dinv_ref[...] = di[:, None]

  return pl.pallas_call(
      body,
      grid=(pl.cdiv(N, BN),),
      in_specs=[
          pl.BlockSpec((BN, F), lambda i: (i, 0)),
          pl.BlockSpec((F, H), lambda i: (0, 0)),
          pl.BlockSpec((NC * NS, BN), lambda i: (0, i)),
      ],
      out_specs=[
          pl.BlockSpec((BN, H), lambda i: (i, 0)),
          pl.BlockSpec((BN, 1), lambda i: (i, 0)),
      ],
      out_shape=[
          jax.ShapeDtypeStruct((N, H), jnp.float32),
          jax.ShapeDtypeStruct((N, 1), jnp.float32),
      ],
  )


def _b2_call(N, H, O, BN):
  def body(accp_ref, xs1_ref, dinv_ref, b1_ref, w2_ref, xs2_ref):
    s = accp_ref[0] + accp_ref[1] + xs1_ref[...]
    h = jnp.maximum(s * dinv_ref[...] + b1_ref[...], 0.0)
    xs2_ref[...] = jnp.dot(h, w2_ref[...],
                           preferred_element_type=jnp.float32) * dinv_ref[...]

  return pl.pallas_call(
      body,
      grid=(pl.cdiv(N, BN),),
      in_specs=[
          pl.BlockSpec((NC, BN, H), lambda i: (0, i, 0)),
          pl.BlockSpec((BN, H), lambda i: (i, 0)),
          pl.BlockSpec((BN, 1), lambda i: (i, 0)),
          pl.BlockSpec((1, H), lambda i: (0, 0)),
          pl.BlockSpec((H, O), lambda i: (0, 0)),
      ],
      out_specs=pl.BlockSpec((BN, O), lambda i: (i, 0)),
      out_shape=jax.ShapeDtypeStruct((N, O), jnp.float32),
  )


def _b3_call(N, O, BN):
  def body(accp_ref, xs2_ref, dinv_ref, b2_ref, out_ref):
    s = accp_ref[0] + accp_ref[1] + xs2_ref[...]
    out_ref[...] = s * dinv_ref[...] + b2_ref[...]

  return pl.pallas_call(
      body,
      grid=(pl.cdiv(N, BN),),
      in_specs=[
          pl.BlockSpec((NC, BN, O), lambda i: (0, i, 0)),
          pl.BlockSpec((BN, O), lambda i: (i, 0)),
          pl.BlockSpec((BN, 1), lambda i: (i, 0)),
          pl.BlockSpec((1, O), lambda i: (0, 0)),
      ],
      out_specs=pl.BlockSpec((BN, O), lambda i: (i, 0)),
      out_shape=jax.ShapeDtypeStruct((N, O), jnp.float32),
  )


def kernel(x, edge_index, edge_attr, W1, b1, W2, b2):
  N, F = x.shape
  H = W1.shape[1]
  O = W2.shape[1]
  E = edge_index.shape[1]
  del edge_attr
  assert E % (NC * NS) == 0 and N % NS == 0
  BN = 1024

  row = edge_index[0]
  col = edge_index[1]
  rpt = _round8(-(-N // NS))
  zerosF = jnp.zeros((rpt, F), jnp.float32)

  nw = NC * NS
  nch = -(-E // (nw * CH))
  nch = (nch + 3) // 4 * 4
  E3 = nw * nch * CH
  NP = rpt * NS
  pad_ar = jnp.arange(E3 - E, dtype=jnp.int32)
  row3 = jnp.concatenate([row, pad_ar % N]).reshape(nw, nch, 1, CH)
  col3 = jnp.concatenate([col, N + pad_ar % (NP - N)]).reshape(nw, nch, 1, CH)
  idx3 = jnp.concatenate([row3, col3], axis=2)

  degp = _deg_call(N, E)(col)
  xs1, dinv = _b1_call(N, F, H, BN)(x, W1, degp)
  accp1 = _edge_call(N, nch, H)(xs1, idx3, zerosF)
  xs2 = _b2_call(N, H, O, BN)(accp1, xs1, dinv, b1.reshape(1, H), W2)
  accp2 = _edge_call(N, nch, O)(xs2, idx3, zerosF)
  out = _b3_call(N, O, BN)(accp2, xs2, dinv, b2.reshape(1, O))
  return out
